# baseline (speedup 1.0000x reference)
import dataclasses
import functools

import jax
import jax.numpy as jnp
from jax import lax
from jax.experimental import pallas as pl
from jax.experimental.pallas import tpu as pltpu
from jax.experimental.pallas import tpu_sc as plsc

_V = 1000000
_D = 64
_B = 4096
_S = 200
_SCALE = 8.0
_NC = 2
_NS = 16
_BT = _B // 128


def _compiler_params():
    cp = pltpu.CompilerParams(use_tc_tiling_on_sc=False)
    if "needs_layout_passes" in pltpu.CompilerParams.__dataclass_fields__:
        cp = dataclasses.replace(cp, needs_layout_passes=False)
    return cp


@jax.jit
def _emb_lookup(table, x_t):
    mesh = plsc.VectorSubcoreMesh(core_axis_name="c", subcore_axis_name="s")

    @functools.partial(
        pl.kernel,
        out_type=jax.ShapeDtypeStruct((_S, _D // 8, _BT, 8, 128), jnp.float32),
        mesh=mesh,
        compiler_params=_compiler_params(),
        scratch_types=[
            pltpu.VMEM((_S, 128), jnp.int32),
            pltpu.VMEM((128, _D), jnp.float32),
            pltpu.VMEM((128, _D), jnp.float32),
            pltpu.VMEM((_D // 8, 8, 128), jnp.float32),
            pltpu.VMEM((_D // 8, 8, 128), jnp.float32),
            pltpu.SemaphoreType.DMA,
            pltpu.SemaphoreType.DMA,
            pltpu.SemaphoreType.DMA,
            pltpu.SemaphoreType.DMA,
        ],
    )
    def k(table_hbm, xt_hbm, out_hbm, idx_all, rows0, rows1, tout0, tout1,
          sg0, sg1, so0, so1):
        wid = lax.axis_index("s") * _NC + lax.axis_index("c")
        bt = wid

        pltpu.sync_copy(xt_hbm.at[:, pl.ds(bt * 128, 128)], idx_all)

        iota = lax.iota(jnp.int32, 16)
        rvecs = [iota + (16 * blk) for blk in range(8)]

        def transpose_scale(rowsb, toutb):
            for dt in range(_D // 8):
                for d8 in range(8):
                    d = dt * 8 + d8
                    cvec = jnp.full((16,), d, jnp.int32)
                    for blk in range(8):
                        vals = plsc.load_gather(rowsb, [rvecs[blk], cvec])
                        toutb.at[dt, d8, pl.ds(16 * blk, 16)][...] = (
                            vals * _SCALE)

        bufs = ((rows0, tout0, sg0, so0), (rows1, tout1, sg1, so1))

        pltpu.async_copy(table_hbm.at[idx_all.at[0]], rows0, sg0)

        @pl.loop(0, _S // 2)
        def _step(i):
            for par in range(2):
                s = i * 2 + par
                rowsb, toutb, sg, so = bufs[par]
                n_rowsb, _, n_sg, _ = bufs[1 - par]

                @pl.when(s + 1 < _S)
                def _():
                    pltpu.async_copy(
                        table_hbm.at[idx_all.at[s + 1]], n_rowsb, n_sg)

                pltpu.make_async_copy(
                    table_hbm.at[idx_all.at[s]], rowsb, sg).wait()

                @pl.when(s >= 2)
                def _():
                    pltpu.make_async_copy(
                        toutb, out_hbm.at[s, :, bt], so).wait()

                transpose_scale(rowsb, toutb)
                pltpu.async_copy(toutb, out_hbm.at[s, :, bt], so)

        pltpu.make_async_copy(tout0, out_hbm.at[0, :, bt], so0).wait()
        pltpu.make_async_copy(tout1, out_hbm.at[0, :, bt], so1).wait()

    return k(table, x_t)


def kernel(x, table):
    x_t = x.astype(jnp.int32).T
    raw = _emb_lookup(table, x_t)
    return raw.transpose(2, 4, 0, 1, 3).reshape(_B, _S, _D)


# speedup vs baseline: 1.4105x; 1.4105x over previous
import dataclasses
import functools

import jax
import jax.numpy as jnp
from jax import lax
from jax.experimental import pallas as pl
from jax.experimental.pallas import tpu as pltpu
from jax.experimental.pallas import tpu_sc as plsc

_V = 1000000
_D = 64
_B = 4096
_S = 200
_SCALE = 8.0
_NC = 2
_NS = 16
_BT = _B // 128


def _compiler_params():
    cp = pltpu.CompilerParams(use_tc_tiling_on_sc=False)
    if "needs_layout_passes" in pltpu.CompilerParams.__dataclass_fields__:
        cp = dataclasses.replace(cp, needs_layout_passes=False)
    return cp


@jax.jit
def _emb_lookup(table, x_t):
    mesh = plsc.VectorSubcoreMesh(core_axis_name="c", subcore_axis_name="s")

    @functools.partial(
        pl.kernel,
        out_type=jax.ShapeDtypeStruct((_S, _D // 8, _BT, 8, 128), jnp.float32),
        mesh=mesh,
        compiler_params=_compiler_params(),
        scratch_types=[
            pltpu.VMEM((_S, 128), jnp.int32),
            pltpu.VMEM((128, _D), jnp.float32),
            pltpu.VMEM((128, _D), jnp.float32),
            pltpu.VMEM((_D // 8, 8, 128), jnp.float32),
            pltpu.VMEM((_D // 8, 8, 128), jnp.float32),
            pltpu.SemaphoreType.DMA,
            pltpu.SemaphoreType.DMA,
            pltpu.SemaphoreType.DMA,
            pltpu.SemaphoreType.DMA,
        ],
    )
    def k(table_hbm, xt_hbm, out_hbm, idx_all, rows0, rows1, tout0, tout1,
          sg0, sg1, so0, so1):
        wid = lax.axis_index("s") * _NC + lax.axis_index("c")
        bt = wid

        pltpu.sync_copy(xt_hbm.at[:, pl.ds(bt * 128, 128)], idx_all)

        iota = lax.iota(jnp.int32, 16)

        def transpose_scale(rowsb, toutb):
            @plsc.parallel_loop(0, (_D * 128) // 16, unroll=8)
            def _t(k):
                rvec = iota + ((k & 7) << 4)
                cvec = lax.broadcast(k >> 3, (16,))
                vals = plsc.load_gather(rowsb, [rvec, cvec])
                toutb.at[k >> 6, (k >> 3) & 7, pl.ds((k & 7) * 16, 16)][
                    ...] = vals * _SCALE

        bufs = ((rows0, tout0, sg0, so0), (rows1, tout1, sg1, so1))

        pltpu.async_copy(table_hbm.at[idx_all.at[0]], rows0, sg0)

        @pl.loop(0, _S // 2)
        def _step(i):
            for par in range(2):
                s = i * 2 + par
                rowsb, toutb, sg, so = bufs[par]
                n_rowsb, _, n_sg, _ = bufs[1 - par]

                @pl.when(s + 1 < _S)
                def _():
                    pltpu.async_copy(
                        table_hbm.at[idx_all.at[s + 1]], n_rowsb, n_sg)

                pltpu.make_async_copy(
                    table_hbm.at[idx_all.at[s]], rowsb, sg).wait()

                @pl.when(s >= 2)
                def _():
                    pltpu.make_async_copy(
                        toutb, out_hbm.at[s, :, bt], so).wait()

                transpose_scale(rowsb, toutb)
                pltpu.async_copy(toutb, out_hbm.at[s, :, bt], so)

        pltpu.make_async_copy(tout0, out_hbm.at[0, :, bt], so0).wait()
        pltpu.make_async_copy(tout1, out_hbm.at[0, :, bt], so1).wait()

    return k(table, x_t)


def kernel(x, table):
    x_t = x.astype(jnp.int32).T
    raw = _emb_lookup(table, x_t)
    return raw.transpose(2, 4, 0, 1, 3).reshape(_B, _S, _D)
